# baseline (speedup 1.0000x reference)
import functools

import jax
import jax.numpy as jnp
from jax import lax
from jax.experimental import pallas as pl
from jax.experimental.pallas import tpu as pltpu

_DOWN_RATIO = 4.0
_K = 100
_BIG = 2**30


def _decode_body(hm_ref, wh_ref, reg_ref, boxes_ref, scores_ref, cls_ref,
                 s_ref, m2_ref, q_ref, *, C, H, W, K):
    h = hm_ref[0]
    heat = jnp.clip(jax.nn.sigmoid(h), 1e-4, 1.0 - 1e-4)

    neg = jnp.float32(-1.0)
    pad_w = jnp.full((C, H, 1), neg, jnp.float32)
    left = jnp.concatenate([pad_w, heat[:, :, : W - 1]], axis=2)
    right = jnp.concatenate([heat[:, :, 1:], pad_w], axis=2)
    hw = jnp.maximum(jnp.maximum(left, right), heat)
    pad_h = jnp.full((C, 1, W), neg, jnp.float32)
    up = jnp.concatenate([pad_h, hw[:, : H - 1, :]], axis=1)
    down = jnp.concatenate([hw[:, 1:, :], pad_h], axis=1)
    hmax = jnp.maximum(jnp.maximum(up, down), hw)

    sup = jnp.where(heat == hmax, heat, 0.0)
    s_ref[...] = sup
    m2_ref[...] = jnp.max(sup, axis=(1, 2))[None]
    q_ref[...] = jnp.zeros_like(q_ref)

    lane = lax.broadcasted_iota(jnp.int32, (1, 128), 1)
    ci = lax.broadcasted_iota(jnp.int32, (1, C), 1)
    flat2 = (lax.broadcasted_iota(jnp.int32, (H, W), 0) * W
             + lax.broadcasted_iota(jnp.int32, (H, W), 1))

    def body(k, carry):
        m2 = m2_ref[...]
        mv = jnp.max(m2, axis=(0, 1), keepdims=True)
        cv = jnp.min(jnp.where(m2 == mv, ci, _BIG), axis=(0, 1),
                     keepdims=True)
        c = cv[0, 0]

        slab = s_ref[c]
        posv = jnp.min(jnp.where(slab == mv, flat2, _BIG), axis=(0, 1),
                       keepdims=True)
        onehot = flat2 == posv
        new_slab = jnp.where(onehot, neg, slab)
        s_ref[c] = new_slab
        cmaxv = jnp.max(new_slab, axis=(0, 1), keepdims=True)
        m2_ref[...] = jnp.where(ci == cv, cmaxv, m2)

        g01 = jnp.sum(jnp.where(onehot[None], reg_ref[0], 0.0),
                      axis=(1, 2))[:, None]
        g23 = jnp.sum(jnp.where(onehot[None], wh_ref[0], 0.0),
                      axis=(1, 2))[:, None]

        rv = (posv // W).astype(jnp.float32)
        colv = (posv % W).astype(jnp.float32)
        vec8 = jnp.concatenate([
            mv, cv.astype(jnp.float32), rv, colv, g01, g23,
        ], axis=0)
        q_ref[...] = jnp.where(lane == k, vec8, q_ref[...])
        return carry

    lax.fori_loop(0, K, body, 0, unroll=False)

    q = q_ref[...]
    score = q[0:1, :K]
    clsv = q[1:2, :K]
    ys = q[2:3, :K] + q[5:6, :K]
    xs = q[3:4, :K] + q[4:5, :K]
    wv = q[6:7, :K]
    hv = q[7:8, :K]
    x1 = (xs - wv * 0.5) * _DOWN_RATIO
    y1 = (ys - hv * 0.5) * _DOWN_RATIO
    x2 = (xs + wv * 0.5) * _DOWN_RATIO
    y2 = (ys + hv * 0.5) * _DOWN_RATIO
    boxes_ref[...] = jnp.concatenate([x1, y1, x2, y2], axis=0)[None]
    scores_ref[...] = score[None]
    cls_ref[...] = clsv[None]


def kernel(hm, wh, reg):
    B, C, H, W = hm.shape
    K = _K
    body = functools.partial(_decode_body, C=C, H=H, W=W, K=K)
    boxes_t, scores, classes = pl.pallas_call(
        body,
        grid=(B,),
        in_specs=[
            pl.BlockSpec((1, C, H, W), lambda b: (b, 0, 0, 0)),
            pl.BlockSpec((1, 2, H, W), lambda b: (b, 0, 0, 0)),
            pl.BlockSpec((1, 2, H, W), lambda b: (b, 0, 0, 0)),
        ],
        out_specs=[
            pl.BlockSpec((1, 4, K), lambda b: (b, 0, 0)),
            pl.BlockSpec((1, 1, K), lambda b: (b, 0, 0)),
            pl.BlockSpec((1, 1, K), lambda b: (b, 0, 0)),
        ],
        out_shape=[
            jax.ShapeDtypeStruct((B, 4, K), jnp.float32),
            jax.ShapeDtypeStruct((B, 1, K), jnp.float32),
            jax.ShapeDtypeStruct((B, 1, K), jnp.float32),
        ],
        scratch_shapes=[
            pltpu.VMEM((C, H, W), jnp.float32),
            pltpu.VMEM((1, C), jnp.float32),
            pltpu.VMEM((8, 128), jnp.float32),
        ],
    )(hm, wh, reg)
    boxes = jnp.transpose(boxes_t, (0, 2, 1))
    return boxes, scores[:, 0, :], classes[:, 0, :]


# speedup vs baseline: 1.0632x; 1.0632x over previous
import functools

import jax
import jax.numpy as jnp
from jax import lax
from jax.experimental import pallas as pl
from jax.experimental.pallas import tpu as pltpu

_DOWN_RATIO = 4.0
_K = 100
_BIG = 2**30


def _decode_body(hm_ref, wh_ref, reg_ref, boxes_ref, scores_ref, cls_ref,
                 s_ref, m2_ref, q_ref, *, C, H, W, K):
    h = hm_ref[0]
    heat = jnp.clip(jax.nn.sigmoid(h), 1e-4, 1.0 - 1e-4)

    neg = jnp.float32(-1.0)
    pad_w = jnp.full((C, H, 1), neg, jnp.float32)
    left = jnp.concatenate([pad_w, heat[:, :, : W - 1]], axis=2)
    right = jnp.concatenate([heat[:, :, 1:], pad_w], axis=2)
    hw = jnp.maximum(jnp.maximum(left, right), heat)
    pad_h = jnp.full((C, 1, W), neg, jnp.float32)
    up = jnp.concatenate([pad_h, hw[:, : H - 1, :]], axis=1)
    down = jnp.concatenate([hw[:, 1:, :], pad_h], axis=1)
    hmax = jnp.maximum(jnp.maximum(up, down), hw)

    sup = jnp.where(heat == hmax, heat, 0.0)
    s_ref[...] = sup
    m2_ref[...] = jnp.max(sup, axis=(1, 2))[None]
    q_ref[...] = jnp.zeros_like(q_ref)

    lane = lax.broadcasted_iota(jnp.int32, (1, 128), 1)
    ci = lax.broadcasted_iota(jnp.int32, (1, C), 1)
    flat2 = (lax.broadcasted_iota(jnp.int32, (H, W), 0) * W
             + lax.broadcasted_iota(jnp.int32, (H, W), 1))

    def body(k, carry):
        m2 = m2_ref[...]
        mv = jnp.max(m2, axis=(0, 1), keepdims=True)
        cv = jnp.min(jnp.where(m2 == mv, ci, _BIG), axis=(0, 1),
                     keepdims=True)
        c = cv[0, 0]

        slab = s_ref[c]
        posv = jnp.min(jnp.where(slab == mv, flat2, _BIG), axis=(0, 1),
                       keepdims=True)
        new_slab = jnp.where(flat2 == posv, neg, slab)
        s_ref[c] = new_slab
        cmaxv = jnp.max(new_slab, axis=(0, 1), keepdims=True)
        m2_ref[...] = jnp.where(ci == cv, cmaxv, m2)

        vec3 = jnp.concatenate([
            mv, cv.astype(jnp.float32), posv.astype(jnp.float32),
        ], axis=0)
        q3 = q_ref[pl.ds(0, 3), :]
        q_ref[pl.ds(0, 3), :] = jnp.where(lane == k, vec3, q3)
        return carry

    lax.fori_loop(0, K, body, 0, unroll=False)

    q = q_ref[...]
    score = q[0:1, :K]
    clsv = q[1:2, :K]
    posi = q[2:3, :].astype(jnp.int32)
    yi = posi // W
    xi = posi % W

    onehot_y = (lax.broadcasted_iota(jnp.int32, (H, 128), 0)
                == jnp.broadcast_to(yi, (H, 128))).astype(jnp.float32)
    onehot_x = (lax.broadcasted_iota(jnp.int32, (W, 128), 0)
                == jnp.broadcast_to(xi, (W, 128))).astype(jnp.float32)

    def gather2(arr):
        t = lax.dot_general(
            arr, onehot_y, (((0,), (0,)), ((), ())),
            precision=lax.Precision.HIGHEST,
            preferred_element_type=jnp.float32)
        return jnp.sum(t * onehot_x, axis=0, keepdims=True)

    g_reg0 = gather2(reg_ref[0, 0])
    g_reg1 = gather2(reg_ref[0, 1])
    g_wh0 = gather2(wh_ref[0, 0])
    g_wh1 = gather2(wh_ref[0, 1])

    ys = yi.astype(jnp.float32)[:, :K] + g_reg1[:, :K]
    xs = xi.astype(jnp.float32)[:, :K] + g_reg0[:, :K]
    wv = g_wh0[:, :K]
    hv = g_wh1[:, :K]
    x1 = (xs - wv * 0.5) * _DOWN_RATIO
    y1 = (ys - hv * 0.5) * _DOWN_RATIO
    x2 = (xs + wv * 0.5) * _DOWN_RATIO
    y2 = (ys + hv * 0.5) * _DOWN_RATIO
    boxes_ref[...] = jnp.concatenate([x1, y1, x2, y2], axis=0)[None]
    scores_ref[...] = score[None]
    cls_ref[...] = clsv[None]


def kernel(hm, wh, reg):
    B, C, H, W = hm.shape
    K = _K
    body = functools.partial(_decode_body, C=C, H=H, W=W, K=K)
    boxes_t, scores, classes = pl.pallas_call(
        body,
        grid=(B,),
        in_specs=[
            pl.BlockSpec((1, C, H, W), lambda b: (b, 0, 0, 0)),
            pl.BlockSpec((1, 2, H, W), lambda b: (b, 0, 0, 0)),
            pl.BlockSpec((1, 2, H, W), lambda b: (b, 0, 0, 0)),
        ],
        out_specs=[
            pl.BlockSpec((1, 4, K), lambda b: (b, 0, 0)),
            pl.BlockSpec((1, 1, K), lambda b: (b, 0, 0)),
            pl.BlockSpec((1, 1, K), lambda b: (b, 0, 0)),
        ],
        out_shape=[
            jax.ShapeDtypeStruct((B, 4, K), jnp.float32),
            jax.ShapeDtypeStruct((B, 1, K), jnp.float32),
            jax.ShapeDtypeStruct((B, 1, K), jnp.float32),
        ],
        scratch_shapes=[
            pltpu.VMEM((C, H, W), jnp.float32),
            pltpu.VMEM((1, C), jnp.float32),
            pltpu.VMEM((8, 128), jnp.float32),
        ],
    )(hm, wh, reg)
    boxes = jnp.transpose(boxes_t, (0, 2, 1))
    return boxes, scores[:, 0, :], classes[:, 0, :]
